# baseline (speedup 1.0000x reference)
import functools

import jax
import jax.numpy as jnp
from jax import lax
from jax.experimental import pallas as pl
from jax.experimental.pallas import tpu as pltpu
from jax.experimental.pallas import tpu_sc as plsc

_NC = 2
_NS = 16
_NW = _NC * _NS
_LANES = 16


@functools.lru_cache(maxsize=None)
def _make_sc_relayout(D, V):
    D2 = 2 * D
    BLKC = 4 * D
    BLKP = BLKC + 1
    ORPB = BLKC // 2
    NBUF = 3
    NT = V // BLKC
    TAILC = V - NT * BLKC
    NI = (NT + _NW - 1) // _NW
    NI += (-NI) % NBUF
    assert D % _LANES == 0

    mesh = plsc.VectorSubcoreMesh(core_axis_name="c", subcore_axis_name="s")

    @functools.partial(
        pl.kernel,
        mesh=mesh,
        compiler_params=pltpu.CompilerParams(needs_layout_passes=False),
        out_type=jax.ShapeDtypeStruct((V // 2, D2), jnp.float32),
        scratch_types=[
            pltpu.VMEM((NBUF, D, BLKC), jnp.float32),
            pltpu.VMEM((D, BLKP), jnp.float32),
            pltpu.VMEM((NBUF, ORPB, D2), jnp.float32),
        ] + [pltpu.SemaphoreType.DMA] * (2 * NBUF),
    )
    def sc_relayout(tabT_hbm, tail_hbm, out_hbm, inb, inp, outb, *sems):
        wid = lax.axis_index("s") * _NC + lax.axis_index("c")
        sis = sems[:NBUF]
        sos = sems[NBUF:]
        rowvecs = [lax.iota(jnp.int32, _LANES) + fc * _LANES
                   for fc in range(D // _LANES)]

        def startin(j, s):
            k = wid + _NW * j
            pltpu.async_copy(
                tabT_hbm.at[:, pl.ds(k * BLKC, BLKC)], inb.at[s], sis[s])

        def xpose(s):
            def cbody(f, carry):
                for cc in range(BLKC // _LANES):
                    inp[f, pl.ds(cc * _LANES, _LANES)] = (
                        inb[s, f, pl.ds(cc * _LANES, _LANES)])
                return carry

            lax.fori_loop(0, D, cbody, 0, unroll=4)

            def rbody(r, carry):
                for p in range(2):
                    col = jnp.zeros((_LANES,), jnp.int32) + (2 * r + p)
                    for fc in range(D // _LANES):
                        v = plsc.load_gather(inp, [rowvecs[fc], col])
                        outb[s, r, pl.ds(p * D + fc * _LANES, _LANES)] = v
                return carry

            lax.fori_loop(0, ORPB, rbody, 0, unroll=4)

        for s in range(NBUF):
            @pl.when(wid + _NW * s < NT)
            def _(s=s):
                startin(s, s)

        def outer(g, carry):
            for s in range(NBUF):
                j = g * NBUF + s
                k = wid + _NW * j

                @pl.when(k < NT)
                def _(j=j, k=k, s=s):
                    pltpu.make_async_copy(
                        tabT_hbm.at[:, pl.ds(k * BLKC, BLKC)],
                        inb.at[s], sis[s]).wait()

                    @pl.when(j >= NBUF)
                    def _():
                        pltpu.make_async_copy(
                            outb.at[s], out_hbm.at[pl.ds(k * ORPB, ORPB)],
                            sos[s]).wait()

                    xpose(s)
                    pltpu.async_copy(
                        outb.at[s], out_hbm.at[pl.ds(k * ORPB, ORPB)], sos[s])
                    kn = wid + _NW * (j + NBUF)

                    @pl.when(kn < NT)
                    def _(j=j, s=s):
                        startin(j + NBUF, s)
            return carry

        lax.fori_loop(0, NI // NBUF, outer, 0)
        for s in range(NBUF):
            pltpu.make_async_copy(
                outb.at[s], out_hbm.at[pl.ds(0, ORPB)], sos[s]).wait()

        @pl.when(wid == _NW - 1)
        def _():
            pltpu.sync_copy(
                tail_hbm, out_hbm.at[pl.ds(NT * ORPB, TAILC // 2)])

    return sc_relayout


@functools.lru_cache(maxsize=None)
def _make_sc_gather_sum(B, L, V2, D):
    D2 = 2 * D
    ROWS = B // _NW
    NBUF = 4
    NVR = D // _LANES
    LPAD = (L + _LANES - 1) // _LANES * _LANES
    ABLK = 64
    GPB = ABLK // NBUF
    assert B % _NW == 0 and ROWS % ABLK == 0 and ABLK % NBUF == 0
    assert L <= 128 and D % _LANES == 0

    mesh = plsc.VectorSubcoreMesh(core_axis_name="c", subcore_axis_name="s")

    @functools.partial(
        pl.kernel,
        mesh=mesh,
        out_type=jax.ShapeDtypeStruct((B, D), jnp.float32),
        scratch_types=[
            pltpu.VMEM((ROWS, LPAD), jnp.int32),
            pltpu.VMEM((NBUF, LPAD), jnp.int32),
            pltpu.VMEM((NBUF, L, D2), jnp.float32),
            pltpu.VMEM((ABLK, D), jnp.float32),
        ] + [pltpu.SemaphoreType.DMA] * NBUF,
    )
    def sc_gather_sum(x64_hbm, tab2_hbm, out_hbm,
                      idx_v, ring_v, rows_v, acc_v, *sems):
        wid = lax.axis_index("s") * _NC + lax.axis_index("c")
        base = wid * ROWS
        pltpu.sync_copy(x64_hbm.at[pl.ds(base, ROWS)], idx_v)

        def start(j, b):
            for grp in range(LPAD // _LANES):
                lo = grp * _LANES
                ring_v[b, pl.ds(lo, _LANES)] = (
                    idx_v[j, pl.ds(lo, _LANES)] >> 1)
            pltpu.async_copy(
                tab2_hbm.at[ring_v.at[b, pl.ds(0, L)]], rows_v.at[b], sems[b])

        for b in range(NBUF):
            start(b, b)

        def outer(g, carry):
            for b in range(NBUF):
                j = g * NBUF + b
                pltpu.make_async_copy(
                    tab2_hbm.at[ring_v.at[b, pl.ds(0, L)]],
                    rows_v.at[b], sems[b]).wait()
                jn = j + NBUF

                @pl.when(jn < ROWS)
                def _():
                    start(jn, b)

                accs = [jnp.zeros((_LANES,), jnp.float32)] * NVR
                for grp in range(LPAD // _LANES):
                    lo = grp * _LANES
                    offv = (idx_v[j, pl.ds(lo, _LANES)] & 1) * D
                    for u in range(min(_LANES, L - lo)):
                        off = offv[u]
                        for q in range(NVR):
                            accs[q] = accs[q] + rows_v[
                                b, lo + u, pl.ds(off + q * _LANES, _LANES)]
                arow = (g % GPB) * NBUF + b
                for q in range(NVR):
                    acc_v[arow, pl.ds(q * _LANES, _LANES)] = accs[q]

            @pl.when(g % GPB == GPB - 1)
            def _():
                blk = g // GPB
                pltpu.sync_copy(
                    acc_v, out_hbm.at[pl.ds(base + blk * ABLK, ABLK)])

            return carry

        lax.fori_loop(0, ROWS // NBUF, outer, 0)

    return sc_gather_sum


@functools.lru_cache(maxsize=None)
def _make_tc_mlp(B, L, D, H, C):
    BLK = 512
    assert B % BLK == 0

    def body(x_ref, s_ref, w1_ref, b1_ref, w2_ref, b2_ref, o_ref):
        cnt = jnp.sum((x_ref[...] != 0).astype(jnp.float32), axis=1,
                      keepdims=True)
        avg = s_ref[...] / jnp.maximum(cnt, 1e-6)
        h = jnp.dot(avg, w1_ref[...], preferred_element_type=jnp.float32)
        h = jnp.maximum(h + b1_ref[...], 0.0)
        o_ref[...] = (jnp.dot(h, w2_ref[...],
                              preferred_element_type=jnp.float32)
                      + b2_ref[...])

    return pl.pallas_call(
        body,
        grid=(B // BLK,),
        in_specs=[
            pl.BlockSpec((BLK, L), lambda i: (i, 0)),
            pl.BlockSpec((BLK, D), lambda i: (i, 0)),
            pl.BlockSpec((D, H), lambda i: (0, 0)),
            pl.BlockSpec((1, H), lambda i: (0, 0)),
            pl.BlockSpec((H, C), lambda i: (0, 0)),
            pl.BlockSpec((1, C), lambda i: (0, 0)),
        ],
        out_specs=pl.BlockSpec((BLK, C), lambda i: (i, 0)),
        out_shape=jax.ShapeDtypeStruct((B, C), jnp.float32),
    )


def kernel(x, table, W1, b1, W2, b2):
    B, L = x.shape
    V, D = table.shape
    H = W1.shape[1]
    C = W2.shape[1]
    xi = x.astype(jnp.int32)
    lpad = (L + 15) // 16 * 16
    x64 = jnp.pad(xi, ((0, 0), (0, lpad - L)))
    ntail = V % (4 * D)
    tail128 = table[V - ntail:].reshape(ntail // 2, 2 * D)
    tab2 = _make_sc_relayout(D, V)(table.T, tail128)
    summed = _make_sc_gather_sum(B, L, V // 2, D)(x64, tab2)
    out = _make_tc_mlp(B, L, D, H, C)(
        xi, summed, W1, b1.reshape(1, H), W2, b2.reshape(1, C))
    return out


# speedup vs baseline: 2.5655x; 2.5655x over previous
import functools

import jax
import jax.numpy as jnp
from jax import lax
from jax.experimental import pallas as pl
from jax.experimental.pallas import tpu as pltpu
from jax.experimental.pallas import tpu_sc as plsc

_NC = 2
_NS = 16
_NW = _NC * _NS
_LANES = 16


@functools.lru_cache(maxsize=None)
def _make_sc_gather_sum(B, L, V, D):
    L2 = 2 * L
    PAIRS = B // 2 // _NW
    ROWS = B // _NW
    NBUF = 4
    NVR = D // _LANES
    ABLK = 64
    GPB = ABLK // (2 * NBUF)
    assert B % (2 * _NW) == 0 and ROWS % ABLK == 0 and ABLK % (2 * NBUF) == 0
    assert L2 <= 128 and D % _LANES == 0

    mesh = plsc.VectorSubcoreMesh(core_axis_name="c", subcore_axis_name="s")

    @functools.partial(
        pl.kernel,
        mesh=mesh,
        compiler_params=pltpu.CompilerParams(use_tc_tiling_on_sc=False),
        out_type=jax.ShapeDtypeStruct((B, D), jnp.float32),
        scratch_types=[
            pltpu.VMEM((PAIRS, L2), jnp.int32),
            pltpu.VMEM((NBUF, L2, D), jnp.float32),
            pltpu.VMEM((ABLK, D), jnp.float32),
        ] + [pltpu.SemaphoreType.DMA] * NBUF,
    )
    def sc_gather_sum(x2_hbm, tab_hbm, out_hbm, idx_v, rows_v, acc_v, *sems):
        wid = lax.axis_index("s") * _NC + lax.axis_index("c")
        pbase = wid * PAIRS
        pltpu.sync_copy(x2_hbm.at[pl.ds(pbase, PAIRS)], idx_v)

        def start(j, b):
            pltpu.async_copy(
                tab_hbm.at[idx_v.at[j]], rows_v.at[b], sems[b])

        for b in range(NBUF):
            start(b, b)

        def outer(g, carry):
            for b in range(NBUF):
                j = g * NBUF + b
                pltpu.make_async_copy(
                    tab_hbm.at[idx_v.at[j]], rows_v.at[b], sems[b]).wait()
                jn = j + NBUF

                @pl.when(jn < PAIRS)
                def _():
                    start(jn, b)

                for half in range(2):
                    accs = [jnp.zeros((_LANES,), jnp.float32)] * NVR
                    for l in range(L):
                        for q in range(NVR):
                            accs[q] = accs[q] + rows_v[
                                b, half * L + l, pl.ds(q * _LANES, _LANES)]
                    arow = ((g % GPB) * NBUF + b) * 2 + half
                    for q in range(NVR):
                        acc_v[arow, pl.ds(q * _LANES, _LANES)] = accs[q]

            @pl.when(g % GPB == GPB - 1)
            def _():
                blk = g // GPB
                pltpu.sync_copy(
                    acc_v,
                    out_hbm.at[pl.ds(wid * ROWS + blk * ABLK, ABLK)])

            return carry

        lax.fori_loop(0, PAIRS // NBUF, outer, 0)

    return sc_gather_sum


@functools.lru_cache(maxsize=None)
def _make_tc_mlp(B, L, D, H, C):
    BLK = 512
    assert B % BLK == 0

    def body(x_ref, s_ref, w1_ref, b1_ref, w2t_ref, b2t_ref, ot_ref):
        cnt = jnp.sum((x_ref[...] != 0).astype(jnp.float32), axis=1,
                      keepdims=True)
        avg = s_ref[...] / jnp.maximum(cnt, 1e-6)
        h = jnp.dot(avg, w1_ref[...], preferred_element_type=jnp.float32)
        h = jnp.maximum(h + b1_ref[...], 0.0)
        ot = lax.dot_general(w2t_ref[...], h, (((1,), (1,)), ((), ())),
                             preferred_element_type=jnp.float32)
        ot_ref[...] = ot + b2t_ref[...]

    return pl.pallas_call(
        body,
        grid=(B // BLK,),
        in_specs=[
            pl.BlockSpec((BLK, L), lambda i: (i, 0)),
            pl.BlockSpec((BLK, D), lambda i: (i, 0)),
            pl.BlockSpec((D, H), lambda i: (0, 0)),
            pl.BlockSpec((1, H), lambda i: (0, 0)),
            pl.BlockSpec((C, H), lambda i: (0, 0)),
            pl.BlockSpec((C, 1), lambda i: (0, 0)),
        ],
        out_specs=pl.BlockSpec((C, BLK), lambda i: (0, i)),
        out_shape=jax.ShapeDtypeStruct((C, B), jnp.float32),
    )


def kernel(x, table, W1, b1, W2, b2):
    B, L = x.shape
    V, D = table.shape
    H = W1.shape[1]
    C = W2.shape[1]
    xi = x.astype(jnp.int32)
    x2 = xi.reshape(B // 2, 2 * L)
    summed = _make_sc_gather_sum(B, L, V, D)(x2, table)
    out_t = _make_tc_mlp(B, L, D, H, C)(
        xi, summed, W1, b1.reshape(1, H), W2.T, b2.reshape(C, 1))
    return out_t.T


# speedup vs baseline: 2.9857x; 1.1638x over previous
import functools

import jax
import jax.numpy as jnp
from jax import lax
from jax.experimental import pallas as pl
from jax.experimental.pallas import tpu as pltpu
from jax.experimental.pallas import tpu_sc as plsc

_NC = 2
_NS = 16
_NW = _NC * _NS
_LANES = 16


@functools.lru_cache(maxsize=None)
def _make_sc_gather_sum(B, L, V, D):
    L2 = 2 * L
    PAIRS = B // 2 // _NW
    ROWS = B // _NW
    NBUF = 4
    NVR = D // _LANES
    UNROLL = 5
    assert B % (2 * _NW) == 0 and PAIRS % NBUF == 0
    assert L2 <= 128 and D % _LANES == 0 and L % UNROLL == 0

    mesh = plsc.VectorSubcoreMesh(core_axis_name="c", subcore_axis_name="s")

    @functools.partial(
        pl.kernel,
        mesh=mesh,
        compiler_params=pltpu.CompilerParams(use_tc_tiling_on_sc=False),
        out_type=jax.ShapeDtypeStruct((B, D), jnp.float32),
        scratch_types=[
            pltpu.VMEM((PAIRS, L2), jnp.int32),
            pltpu.VMEM((NBUF, L2, D), jnp.float32),
            pltpu.VMEM((ROWS, D), jnp.float32),
        ] + [pltpu.SemaphoreType.DMA] * NBUF,
    )
    def sc_gather_sum(x2_hbm, table_hbm, out_hbm, idx_v, rows_v, acc_v, *sems):
        wid = lax.axis_index("s") * _NC + lax.axis_index("c")
        pbase = wid * PAIRS
        pltpu.sync_copy(x2_hbm.at[pl.ds(pbase, PAIRS)], idx_v)

        def start(j, b):
            pltpu.async_copy(table_hbm.at[idx_v.at[j]], rows_v.at[b], sems[b])

        for b in range(NBUF):
            start(b, b)

        def outer(g, carry):
            for b in range(NBUF):
                j = g * NBUF + b
                pltpu.make_async_copy(
                    table_hbm.at[idx_v.at[j]], rows_v.at[b], sems[b]).wait()
                jn = j + NBUF

                @pl.when(jn < PAIRS)
                def _():
                    start(jn, b)

                for half in range(2):
                    def body(i, accs, half=half, b=b):
                        out = list(accs)
                        for u in range(UNROLL):
                            r = half * L + i * UNROLL + u
                            for q in range(NVR):
                                out[q] = out[q] + rows_v[
                                    b, r, pl.ds(q * _LANES, _LANES)]
                        return tuple(out)

                    zero = jnp.zeros((_LANES,), jnp.float32)
                    accs = lax.fori_loop(0, L // UNROLL, body, (zero,) * NVR)
                    row = 2 * j + half
                    for q in range(NVR):
                        acc_v[row, pl.ds(q * _LANES, _LANES)] = accs[q]
            return carry

        lax.fori_loop(0, PAIRS // NBUF, outer, 0)
        pltpu.sync_copy(acc_v, out_hbm.at[pl.ds(wid * ROWS, ROWS)])

    return sc_gather_sum


@functools.lru_cache(maxsize=None)
def _make_tc_mlp(B, L, D, H, C):
    BLK = 512
    assert B % BLK == 0

    def body(x_ref, s_ref, w1_ref, b1_ref, w2t_ref, b2t_ref, ot_ref):
        cnt = jnp.sum((x_ref[...] != 0).astype(jnp.float32), axis=1,
                      keepdims=True)
        avg = s_ref[...] / jnp.maximum(cnt, 1e-6)
        h = jnp.dot(avg, w1_ref[...], preferred_element_type=jnp.float32)
        h = jnp.maximum(h + b1_ref[...], 0.0)
        ot = lax.dot_general(w2t_ref[...], h, (((1,), (1,)), ((), ())),
                             preferred_element_type=jnp.float32)
        ot_ref[...] = ot + b2t_ref[...]

    return pl.pallas_call(
        body,
        grid=(B // BLK,),
        in_specs=[
            pl.BlockSpec((BLK, L), lambda i: (i, 0)),
            pl.BlockSpec((BLK, D), lambda i: (i, 0)),
            pl.BlockSpec((D, H), lambda i: (0, 0)),
            pl.BlockSpec((1, H), lambda i: (0, 0)),
            pl.BlockSpec((C, H), lambda i: (0, 0)),
            pl.BlockSpec((C, 1), lambda i: (0, 0)),
        ],
        out_specs=pl.BlockSpec((C, BLK), lambda i: (0, i)),
        out_shape=jax.ShapeDtypeStruct((C, B), jnp.float32),
    )


def kernel(x, table, W1, b1, W2, b2):
    B, L = x.shape
    V, D = table.shape
    H = W1.shape[1]
    C = W2.shape[1]
    xi = x.astype(jnp.int32)
    x2 = xi.reshape(B // 2, 2 * L)
    summed = _make_sc_gather_sum(B, L, V, D)(x2, table)
    out_t = _make_tc_mlp(B, L, D, H, C)(
        xi, summed, W1, b1.reshape(1, H), W2.T, b2.reshape(C, 1))
    return out_t.T


# speedup vs baseline: 3.0029x; 1.0058x over previous
import functools

import jax
import jax.numpy as jnp
from jax import lax
from jax.experimental import pallas as pl
from jax.experimental.pallas import tpu as pltpu
from jax.experimental.pallas import tpu_sc as plsc

_NC = 2
_NS = 16
_NW = _NC * _NS
_LANES = 16


@functools.lru_cache(maxsize=None)
def _make_sc_gather_sum(B, L, V, D):
    L2 = 2 * L
    PAIRS = B // 2 // _NW
    ROWS = B // _NW
    NBUF = 8
    NVR = D // _LANES
    UNROLL = 5
    assert B % (2 * _NW) == 0 and PAIRS % NBUF == 0
    assert L2 <= 128 and D % _LANES == 0 and L % UNROLL == 0

    mesh = plsc.VectorSubcoreMesh(core_axis_name="c", subcore_axis_name="s")

    @functools.partial(
        pl.kernel,
        mesh=mesh,
        compiler_params=pltpu.CompilerParams(use_tc_tiling_on_sc=False),
        out_type=jax.ShapeDtypeStruct((B, D), jnp.float32),
        scratch_types=[
            pltpu.VMEM((PAIRS, L2), jnp.int32),
            pltpu.VMEM((NBUF, L2, D), jnp.float32),
            pltpu.VMEM((ROWS, D), jnp.float32),
        ] + [pltpu.SemaphoreType.DMA] * NBUF,
    )
    def sc_gather_sum(x2_hbm, table_hbm, out_hbm, idx_v, rows_v, acc_v, *sems):
        wid = lax.axis_index("s") * _NC + lax.axis_index("c")
        pbase = wid * PAIRS
        pltpu.sync_copy(x2_hbm.at[pl.ds(pbase, PAIRS)], idx_v)

        def start(j, b):
            pltpu.async_copy(table_hbm.at[idx_v.at[j]], rows_v.at[b], sems[b])

        for b in range(NBUF):
            start(b, b)

        def outer(g, carry):
            for b in range(NBUF):
                j = g * NBUF + b
                pltpu.make_async_copy(
                    table_hbm.at[idx_v.at[j]], rows_v.at[b], sems[b]).wait()
                jn = j + NBUF

                @pl.when(jn < PAIRS)
                def _():
                    start(jn, b)

                for half in range(2):
                    def body(i, accs, half=half, b=b):
                        out = list(accs)
                        for u in range(UNROLL):
                            r = half * L + i * UNROLL + u
                            for q in range(NVR):
                                out[q] = out[q] + rows_v[
                                    b, r, pl.ds(q * _LANES, _LANES)]
                        return tuple(out)

                    zero = jnp.zeros((_LANES,), jnp.float32)
                    accs = lax.fori_loop(0, L // UNROLL, body, (zero,) * NVR)
                    row = 2 * j + half
                    for q in range(NVR):
                        acc_v[row, pl.ds(q * _LANES, _LANES)] = accs[q]
            return carry

        lax.fori_loop(0, PAIRS // NBUF, outer, 0)
        pltpu.sync_copy(acc_v, out_hbm.at[pl.ds(wid * ROWS, ROWS)])

    return sc_gather_sum


@functools.lru_cache(maxsize=None)
def _make_tc_mlp(B, L, D, H, C):
    BLK = 512
    assert B % BLK == 0

    def body(x_ref, s_ref, w1_ref, b1_ref, w2t_ref, b2t_ref, ot_ref):
        cnt = jnp.sum((x_ref[...] != 0).astype(jnp.float32), axis=1,
                      keepdims=True)
        avg = s_ref[...] / jnp.maximum(cnt, 1e-6)
        h = jnp.dot(avg, w1_ref[...], preferred_element_type=jnp.float32)
        h = jnp.maximum(h + b1_ref[...], 0.0)
        ot = lax.dot_general(w2t_ref[...], h, (((1,), (1,)), ((), ())),
                             preferred_element_type=jnp.float32)
        ot_ref[...] = ot + b2t_ref[...]

    return pl.pallas_call(
        body,
        grid=(B // BLK,),
        in_specs=[
            pl.BlockSpec((BLK, L), lambda i: (i, 0)),
            pl.BlockSpec((BLK, D), lambda i: (i, 0)),
            pl.BlockSpec((D, H), lambda i: (0, 0)),
            pl.BlockSpec((1, H), lambda i: (0, 0)),
            pl.BlockSpec((C, H), lambda i: (0, 0)),
            pl.BlockSpec((C, 1), lambda i: (0, 0)),
        ],
        out_specs=pl.BlockSpec((C, BLK), lambda i: (0, i)),
        out_shape=jax.ShapeDtypeStruct((C, B), jnp.float32),
    )


def kernel(x, table, W1, b1, W2, b2):
    B, L = x.shape
    V, D = table.shape
    H = W1.shape[1]
    C = W2.shape[1]
    xi = x.astype(jnp.int32)
    x2 = xi.reshape(B // 2, 2 * L)
    summed = _make_sc_gather_sum(B, L, V, D)(x2, table)
    out_t = _make_tc_mlp(B, L, D, H, C)(
        xi, summed, W1, b1.reshape(1, H), W2.T, b2.reshape(C, 1))
    return out_t.T
